# baseline (speedup 1.0000x reference)
import functools

import jax
import jax.numpy as jnp
from jax import lax
from jax.experimental import pallas as pl
from jax.experimental.pallas import tpu as pltpu
from jax.experimental.pallas import tpu_sc as plsc

B = 16
T = 16384
D = 128
NR = 5
NW = 32
TH = T // 2
ROWS_PER_W = TH // NW
IDX_CH = 128
N_CH = ROWS_PER_W // IDX_CH
NEG = -3e38

_NT = (((1,), (1,)), ((), ()))
_TN = (((0,), (0,)), ((), ()))


def _sc_gather_qj(users2d, nodes_v, utab, itab):
  mesh = plsc.VectorSubcoreMesh(core_axis_name="c", subcore_axis_name="s")

  @functools.partial(
      pl.kernel,
      mesh=mesh,
      out_type=(
          jax.ShapeDtypeStruct((TH, D), jnp.float32),
          jax.ShapeDtypeStruct((B, D), jnp.float32),
      ),
      scratch_types=[
          pltpu.VMEM((N_CH, IDX_CH), jnp.int32),
          pltpu.VMEM((ROWS_PER_W, D), jnp.float32),
          pltpu.VMEM((B,), jnp.int32),
          pltpu.VMEM((B, D), jnp.float32),
          pltpu.SemaphoreType.DMA,
          pltpu.SemaphoreType.DMA,
      ],
  )
  def k(users_hbm, nodes_hbm, utab_hbm, itab_hbm, pt_hbm, qj_hbm,
        idx_v, rows_v, nidx_v, qrows_v, sem, qsem):
    wid = lax.axis_index("s") * 2 + lax.axis_index("c")
    pltpu.sync_copy(users_hbm.at[pl.ds(wid * N_CH, N_CH)], idx_v)
    copies = [
        pltpu.async_copy(utab_hbm.at[idx_v.at[c]],
                         rows_v.at[pl.ds(c * IDX_CH, IDX_CH)], sem)
        for c in range(N_CH)
    ]

    @pl.when(wid == 0)
    def _():
      pltpu.sync_copy(nodes_hbm, nidx_v)
      pltpu.async_copy(itab_hbm.at[nidx_v], qrows_v, qsem).wait()
      pltpu.sync_copy(qrows_v, qj_hbm)

    for cp in copies:
      cp.wait()
    pltpu.sync_copy(rows_v, pt_hbm.at[pl.ds(wid * ROWS_PER_W, ROWS_PER_W)])

  return k(users2d, nodes_v, utab, itab)


def _sc_gather(users2d, utab):
  mesh = plsc.VectorSubcoreMesh(core_axis_name="c", subcore_axis_name="s")

  @functools.partial(
      pl.kernel,
      mesh=mesh,
      out_type=jax.ShapeDtypeStruct((TH, D), jnp.float32),
      scratch_types=[
          pltpu.VMEM((N_CH, IDX_CH), jnp.int32),
          pltpu.VMEM((ROWS_PER_W, D), jnp.float32),
          pltpu.SemaphoreType.DMA,
      ],
  )
  def k(users_hbm, utab_hbm, pt_hbm, idx_v, rows_v, sem):
    wid = lax.axis_index("s") * 2 + lax.axis_index("c")
    pltpu.sync_copy(users_hbm.at[pl.ds(wid * N_CH, N_CH)], idx_v)
    copies = [
        pltpu.async_copy(utab_hbm.at[idx_v.at[c]],
                         rows_v.at[pl.ds(c * IDX_CH, IDX_CH)], sem)
        for c in range(N_CH)
    ]
    for cp in copies:
      cp.wait()
    pltpu.sync_copy(rows_v, pt_hbm.at[pl.ds(wid * ROWS_PER_W, ROWS_PER_W)])

  return k(users2d, utab)


def _tc_half_body(pt_ref, ratr_ref, segr_ref, qj_ref, r5_ref,
                  g1_ref, g1b_ref, g2_ref, g2b_ref,
                  a1_ref, a1b_ref, a2_ref, a2b_ref,
                  a3_ref, a3b_ref, m_ref, d_ref, z_ref):
  f32 = jnp.float32
  bf16 = jnp.bfloat16

  r1 = (lax.dot_general(r5_ref[...], g1_ref[:, D:], _NT,
                        preferred_element_type=f32) + g1b_ref[...])
  r1 = jnp.concatenate([r1, jnp.zeros((8 - NR, D), f32)], axis=0).astype(bf16)
  s1 = (lax.dot_general(qj_ref[...], a1_ref[:, D:], _NT,
                        preferred_element_type=f32) + a1b_ref[...]).astype(bf16)

  pt = pt_ref[...].astype(bf16)
  ratr = ratr_ref[0:1, :]
  segr = segr_ref[0:1, :]

  rohT = (lax.broadcasted_iota(jnp.int32, (8, TH), 0) == ratr).astype(bf16)
  sohT = lax.broadcasted_iota(jnp.int32, (B, TH), 0) == segr
  sohTbf = sohT.astype(bf16)

  h = jnp.maximum(
      lax.dot_general(pt, g1_ref[:, :D].astype(bf16), _NT,
                      preferred_element_type=f32)
      + lax.dot_general(rohT, r1, _TN, preferred_element_type=f32),
      0.0).astype(bf16)
  fjt = jnp.maximum(
      lax.dot_general(h, g2_ref[...].astype(bf16), _NT,
                      preferred_element_type=f32)
      + g2b_ref[...], 0.0).astype(bf16)
  a = jnp.maximum(
      lax.dot_general(fjt, a1_ref[:, :D].astype(bf16), _NT,
                      preferred_element_type=f32)
      + lax.dot_general(sohTbf, s1, _TN, preferred_element_type=f32),
      0.0).astype(bf16)
  a = jnp.maximum(
      lax.dot_general(a, a2_ref[...].astype(bf16), _NT,
                      preferred_element_type=f32)
      + a2b_ref[...], 0.0).astype(bf16)
  s_row = lax.dot_general(a3_ref[...].astype(bf16), a, _NT,
                          preferred_element_type=f32) + a3b_ref[0, 0]

  bm = jnp.max(jnp.where(sohT, s_row, NEG), axis=1, keepdims=True)
  m_tok = jnp.max(jnp.where(sohT, bm, NEG), axis=0, keepdims=True)
  e_row = jnp.exp(s_row - m_tok)
  w = jnp.where(sohT, e_row, 0.0)
  m_ref[...] = bm
  d_ref[...] = jnp.sum(w, axis=1, keepdims=True)
  z_ref[...] = jnp.dot(w.astype(bf16), fjt, preferred_element_type=f32)


def _combine_body(m0_ref, d0_ref, z0_ref, m1_ref, d1_ref, z1_ref, out_ref):
  m0, m1 = m0_ref[...], m1_ref[...]
  m = jnp.maximum(m0, m1)
  e0 = jnp.exp(m0 - m)
  e1 = jnp.exp(m1 - m)
  d = d0_ref[...] * e0 + d1_ref[...] * e1
  z = z0_ref[...] * e0 + z1_ref[...] * e1
  out_ref[...] = z / jnp.where(d > 0, d, 1.0)


def _tc_half(pt, ratr, segr, qj, embed_r_w,
             g1_w, g1_b, g2_w, g2_b, a1_w, a1_b, a2_w, a2_b, a3_w, a3_b):
  return pl.pallas_call(
      _tc_half_body,
      out_shape=(
          jax.ShapeDtypeStruct((B, 1), jnp.float32),
          jax.ShapeDtypeStruct((B, 1), jnp.float32),
          jax.ShapeDtypeStruct((B, D), jnp.float32),
      ),
  )(pt, ratr, segr, qj, embed_r_w,
    g1_w, g1_b.reshape(1, D), g2_w, g2_b.reshape(1, D),
    a1_w, a1_b.reshape(1, D), a2_w, a2_b.reshape(1, D),
    a3_w, a3_b.reshape(1, 1))


def kernel(nodes_v, flat_users, flat_ratings, segment_ids, embed_u_w,
           embed_i_w, embed_r_w, g1_w, g1_b, g2_w, g2_b, a1_w, a1_b,
           a2_w, a2_b, a3_w, a3_b):
  users2d = flat_users.reshape(T // IDX_CH, IDX_CH)
  nrow = TH // IDX_CH
  pt0, qj = _sc_gather_qj(users2d[:nrow], nodes_v, embed_u_w, embed_i_w)
  pt1 = _sc_gather(users2d[nrow:], embed_u_w)

  rat2 = flat_ratings.reshape(2, TH)
  seg2 = segment_ids.reshape(2, TH)
  wargs = (qj, embed_r_w, g1_w, g1_b, g2_w, g2_b,
           a1_w, a1_b, a2_w, a2_b, a3_w, a3_b)
  m0, d0, z0 = _tc_half(pt0, rat2[0:1], seg2[0:1], *wargs)
  m1, d1, z1 = _tc_half(pt1, rat2[1:2], seg2[1:2], *wargs)

  return pl.pallas_call(
      _combine_body,
      out_shape=jax.ShapeDtypeStruct((B, D), jnp.float32),
  )(m0, d0, z0, m1, d1, z1)


# speedup vs baseline: 1.0380x; 1.0380x over previous
import functools

import jax
import jax.numpy as jnp
from jax import lax
from jax.experimental import pallas as pl
from jax.experimental.pallas import tpu as pltpu
from jax.experimental.pallas import tpu_sc as plsc

B = 16
T = 16384
D = 128
NR = 5
NW = 32
TH = T // 2
ROWS_PER_W = TH // NW
IDX_CH = 128
N_CH = ROWS_PER_W // IDX_CH
NEG = -3e38

_NT = (((1,), (1,)), ((), ()))
_TN = (((0,), (0,)), ((), ()))


def _sc_gather_qj(users2d, nodes_v, utab, itab):
  mesh = plsc.VectorSubcoreMesh(core_axis_name="c", subcore_axis_name="s")

  @functools.partial(
      pl.kernel,
      mesh=mesh,
      out_type=(
          jax.ShapeDtypeStruct((TH, D), jnp.float32),
          jax.ShapeDtypeStruct((B, D), jnp.float32),
      ),
      scratch_types=[
          pltpu.VMEM((N_CH, IDX_CH), jnp.int32),
          pltpu.VMEM((ROWS_PER_W, D), jnp.float32),
          pltpu.VMEM((B,), jnp.int32),
          pltpu.VMEM((B, D), jnp.float32),
          pltpu.SemaphoreType.DMA,
          pltpu.SemaphoreType.DMA,
      ],
  )
  def k(users_hbm, nodes_hbm, utab_hbm, itab_hbm, pt_hbm, qj_hbm,
        idx_v, rows_v, nidx_v, qrows_v, sem, qsem):
    wid = lax.axis_index("s") * 2 + lax.axis_index("c")
    pltpu.sync_copy(users_hbm.at[pl.ds(wid * N_CH, N_CH)], idx_v)
    copies = [
        pltpu.async_copy(utab_hbm.at[idx_v.at[c]],
                         rows_v.at[pl.ds(c * IDX_CH, IDX_CH)], sem)
        for c in range(N_CH)
    ]

    @pl.when(wid == 0)
    def _():
      pltpu.sync_copy(nodes_hbm, nidx_v)
      pltpu.async_copy(itab_hbm.at[nidx_v], qrows_v, qsem).wait()
      pltpu.sync_copy(qrows_v, qj_hbm)

    for cp in copies:
      cp.wait()
    pltpu.sync_copy(rows_v, pt_hbm.at[pl.ds(wid * ROWS_PER_W, ROWS_PER_W)])

  return k(users2d, nodes_v, utab, itab)


def _sc_gather(users2d, utab, half):
  mesh = plsc.VectorSubcoreMesh(core_axis_name="c", subcore_axis_name="s")
  row0 = half * (TH // IDX_CH)

  @functools.partial(
      pl.kernel,
      mesh=mesh,
      out_type=jax.ShapeDtypeStruct((TH, D), jnp.float32),
      scratch_types=[
          pltpu.VMEM((N_CH, IDX_CH), jnp.int32),
          pltpu.VMEM((ROWS_PER_W, D), jnp.float32),
          pltpu.SemaphoreType.DMA,
      ],
  )
  def k(users_hbm, utab_hbm, pt_hbm, idx_v, rows_v, sem):
    wid = lax.axis_index("s") * 2 + lax.axis_index("c")
    pltpu.sync_copy(users_hbm.at[pl.ds(row0 + wid * N_CH, N_CH)], idx_v)
    copies = [
        pltpu.async_copy(utab_hbm.at[idx_v.at[c]],
                         rows_v.at[pl.ds(c * IDX_CH, IDX_CH)], sem)
        for c in range(N_CH)
    ]
    for cp in copies:
      cp.wait()
    pltpu.sync_copy(rows_v, pt_hbm.at[pl.ds(wid * ROWS_PER_W, ROWS_PER_W)])

  return k(users2d, utab)


def _tc_half_body(final, pt_ref, ratr_ref, segr_ref, qj_ref, r5_ref,
                  g1_ref, g1b_ref, g2_ref, g2b_ref,
                  a1_ref, a1b_ref, a2_ref, a2b_ref,
                  a3_ref, a3b_ref, *rest):
  f32 = jnp.float32
  bf16 = jnp.bfloat16

  r1 = (lax.dot_general(r5_ref[...], g1_ref[:, D:], _NT,
                        preferred_element_type=f32) + g1b_ref[...])
  r1 = jnp.concatenate([r1, jnp.zeros((8 - NR, D), f32)], axis=0).astype(bf16)
  s1 = (lax.dot_general(qj_ref[...], a1_ref[:, D:], _NT,
                        preferred_element_type=f32) + a1b_ref[...]).astype(bf16)

  pt = pt_ref[...].astype(bf16)
  ratr = ratr_ref[0, 0:1, :]
  segr = segr_ref[0, 0:1, :]

  rohT = (lax.broadcasted_iota(jnp.int32, (8, TH), 0) == ratr).astype(bf16)
  sohT = lax.broadcasted_iota(jnp.int32, (B, TH), 0) == segr
  sohTbf = sohT.astype(bf16)

  h = jnp.maximum(
      lax.dot_general(pt, g1_ref[:, :D].astype(bf16), _NT,
                      preferred_element_type=f32)
      + lax.dot_general(rohT, r1, _TN, preferred_element_type=f32),
      0.0).astype(bf16)
  fjt = jnp.maximum(
      lax.dot_general(h, g2_ref[...].astype(bf16), _NT,
                      preferred_element_type=f32)
      + g2b_ref[...], 0.0).astype(bf16)
  a = jnp.maximum(
      lax.dot_general(fjt, a1_ref[:, :D].astype(bf16), _NT,
                      preferred_element_type=f32)
      + lax.dot_general(sohTbf, s1, _TN, preferred_element_type=f32),
      0.0).astype(bf16)
  a = jnp.maximum(
      lax.dot_general(a, a2_ref[...].astype(bf16), _NT,
                      preferred_element_type=f32)
      + a2b_ref[...], 0.0).astype(bf16)
  s_row = lax.dot_general(a3_ref[...].astype(bf16), a, _NT,
                          preferred_element_type=f32) + a3b_ref[0, 0]

  bm = jnp.max(jnp.where(sohT, s_row, NEG), axis=1, keepdims=True)
  m_tok = jnp.max(jnp.where(sohT, bm, NEG), axis=0, keepdims=True)
  e_row = jnp.exp(s_row - m_tok)
  w = jnp.where(sohT, e_row, 0.0)
  d = jnp.sum(w, axis=1, keepdims=True)
  z = jnp.dot(w.astype(bf16), fjt, preferred_element_type=f32)

  if final:
    m0_ref, d0_ref, z0_ref, out_ref = rest
    m0 = m0_ref[...]
    m = jnp.maximum(m0, bm)
    e0 = jnp.exp(m0 - m)
    e1 = jnp.exp(bm - m)
    dd = d0_ref[...] * e0 + d * e1
    zz = z0_ref[...] * e0 + z * e1
    out_ref[...] = zz / jnp.where(dd > 0, dd, 1.0)
  else:
    m_ref, d_ref, z_ref = rest
    m_ref[...] = bm
    d_ref[...] = d
    z_ref[...] = z


def _tc_half(half, pt, rat2, seg2, prev, qj, embed_r_w,
             g1_w, g1_b, g2_w, g2_b, a1_w, a1_b, a2_w, a2_b, a3_w, a3_b):
  final = prev is not None
  full = lambda shape: pl.BlockSpec(shape, lambda i: (0,) * len(shape))
  in_specs = [
      full((TH, D)),
      pl.BlockSpec((1, 1, TH), lambda i: (half, 0, 0)),
      pl.BlockSpec((1, 1, TH), lambda i: (half, 0, 0)),
      full((B, D)),
      full((NR, D)),
      full((D, 2 * D)),
      full((1, D)),
      full((D, D)),
      full((1, D)),
      full((D, 2 * D)),
      full((1, D)),
      full((D, D)),
      full((1, D)),
      full((1, D)),
      full((1, 1)),
  ]
  args = [pt, rat2, seg2, qj, embed_r_w,
          g1_w, g1_b.reshape(1, D), g2_w, g2_b.reshape(1, D),
          a1_w, a1_b.reshape(1, D), a2_w, a2_b.reshape(1, D),
          a3_w, a3_b.reshape(1, 1)]
  if final:
    in_specs += [full((B, 1)), full((B, 1)), full((B, D))]
    args += list(prev)
    out_shape = jax.ShapeDtypeStruct((B, D), jnp.float32)
    out_specs = pl.BlockSpec((B, D), lambda i: (0, 0))
  else:
    out_shape = (
        jax.ShapeDtypeStruct((B, 1), jnp.float32),
        jax.ShapeDtypeStruct((B, 1), jnp.float32),
        jax.ShapeDtypeStruct((B, D), jnp.float32),
    )
    out_specs = (pl.BlockSpec((B, 1), lambda i: (0, 0)),
                 pl.BlockSpec((B, 1), lambda i: (0, 0)),
                 pl.BlockSpec((B, D), lambda i: (0, 0)))
  return pl.pallas_call(
      functools.partial(_tc_half_body, final),
      grid=(1,),
      in_specs=in_specs,
      out_specs=out_specs,
      out_shape=out_shape,
  )(*args)


def kernel(nodes_v, flat_users, flat_ratings, segment_ids, embed_u_w,
           embed_i_w, embed_r_w, g1_w, g1_b, g2_w, g2_b, a1_w, a1_b,
           a2_w, a2_b, a3_w, a3_b):
  users2d = flat_users.reshape(T // IDX_CH, IDX_CH)
  pt0, qj = _sc_gather_qj(users2d, nodes_v, embed_u_w, embed_i_w)
  pt1 = _sc_gather(users2d, embed_u_w, half=1)

  rat2 = flat_ratings.reshape(2, 1, TH)
  seg2 = segment_ids.reshape(2, 1, TH)
  wargs = (qj, embed_r_w, g1_w, g1_b, g2_w, g2_b,
           a1_w, a1_b, a2_w, a2_b, a3_w, a3_b)
  part0 = _tc_half(0, pt0, rat2, seg2, None, *wargs)
  return _tc_half(1, pt1, rat2, seg2, part0, *wargs)
